# baseline (speedup 1.0000x reference)
import functools

import jax
import jax.numpy as jnp
from jax import lax
from jax.experimental import pallas as pl
from jax.experimental.pallas import tpu as pltpu
from jax.experimental.pallas import tpu_sc as plsc

VOCAB = 100000
EMBED_DIM = 64
BATCH = 4096
NC = 2
NS = 16
NW = NC * NS
B_PER_W = BATCH // NW
LANES = 16
CHUNKS = EMBED_DIM // LANES


def _sc_partial_sums(table, head, tail):
    mesh = plsc.VectorSubcoreMesh(core_axis_name="c", subcore_axis_name="s",
                                  num_cores=NC, num_subcores=NS)

    @functools.partial(
        pl.kernel,
        out_type=jax.ShapeDtypeStruct((NW, LANES), jnp.float32),
        mesh=mesh,
        compiler_params=pltpu.CompilerParams(use_tc_tiling_on_sc=False),
        scratch_types=[
            pltpu.VMEM((B_PER_W,), jnp.int32),
            pltpu.VMEM((B_PER_W,), jnp.int32),
            pltpu.VMEM((B_PER_W, EMBED_DIM), jnp.float32),
            pltpu.VMEM((B_PER_W, EMBED_DIM), jnp.float32),
            pltpu.VMEM((LANES,), jnp.float32),
            pltpu.SemaphoreType.DMA,
        ],
    )
    def k(table_hbm, head_hbm, tail_hbm, out_hbm,
          hidx_v, tidx_v, hrows_v, trows_v, acc_v, sem):
        wid = lax.axis_index("s") * NC + lax.axis_index("c")
        base = wid * B_PER_W
        pltpu.sync_copy(head_hbm.at[pl.ds(base, B_PER_W)], hidx_v)
        pltpu.sync_copy(tail_hbm.at[pl.ds(base, B_PER_W)], tidx_v)
        ch = pltpu.async_copy(table_hbm.at[hidx_v], hrows_v, sem)
        ct = pltpu.async_copy(table_hbm.at[tidx_v], trows_v, sem)
        ch.wait()
        ct.wait()

        def body(i, accs):
            new = []
            for j in range(CHUNKS):
                h = hrows_v[i, pl.ds(j * LANES, LANES)]
                t = trows_v[i, pl.ds(j * LANES, LANES)]
                d = h - t
                new.append(accs[j] + d * d)
            return tuple(new)

        zero = jnp.zeros((LANES,), jnp.float32)
        accs = lax.fori_loop(0, B_PER_W, body, (zero,) * CHUNKS)
        acc_v[...] = accs[0] + accs[1] + accs[2] + accs[3]
        pltpu.sync_copy(acc_v, out_hbm.at[wid])

    return k(table, head, tail)


def kernel(head, tail, table):
    partials = _sc_partial_sums(table, head.astype(jnp.int32),
                                tail.astype(jnp.int32))
    return jnp.sqrt(jnp.sum(partials))


# speedup vs baseline: 2.2677x; 2.2677x over previous
import functools

import jax
import jax.numpy as jnp
from jax import lax
from jax.experimental import pallas as pl
from jax.experimental.pallas import tpu as pltpu
from jax.experimental.pallas import tpu_sc as plsc

VOCAB = 100000
EMBED_DIM = 64
BATCH = 4096
NC = 2
NS = 16
NW = NC * NS
COLS_PER_W = EMBED_DIM // NW
LANES = 16
N_CHUNKS = BATCH // LANES
SPLIT = 50048
REST = VOCAB - SPLIT


def _sc_col_partials(table_t, head, tail):
    mesh = plsc.VectorSubcoreMesh(core_axis_name="c", subcore_axis_name="s",
                                  num_cores=NC, num_subcores=NS)

    @functools.partial(
        pl.kernel,
        out_type=jax.ShapeDtypeStruct((NW, LANES), jnp.float32),
        mesh=mesh,
        compiler_params=pltpu.CompilerParams(use_tc_tiling_on_sc=True,
                                             needs_layout_passes=False),
        scratch_types=[
            pltpu.VMEM((BATCH,), jnp.int32),
            pltpu.VMEM((BATCH,), jnp.int32),
            pltpu.VMEM((SPLIT,), jnp.float32),
            pltpu.VMEM((REST,), jnp.float32),
            pltpu.VMEM((BATCH,), jnp.float32),
            pltpu.VMEM((BATCH,), jnp.float32),
            pltpu.VMEM((LANES,), jnp.float32),
            pltpu.SemaphoreType.DMA,
            pltpu.SemaphoreType.DMA,
        ],
    )
    def k(tab_hbm, head_hbm, tail_hbm, out_hbm,
          hidx_v, tidx_v, lo_v, hi_v, hval_v, tval_v, acc_v, sem_a, sem_b):
        wid = lax.axis_index("s") * NC + lax.axis_index("c")
        c0 = wid * COLS_PER_W
        c1 = c0 + 1

        d_lo = pltpu.async_copy(tab_hbm.at[c0, pl.ds(0, SPLIT)], lo_v, sem_a)
        d_hi = pltpu.async_copy(tab_hbm.at[c0, pl.ds(SPLIT, REST)], hi_v, sem_b)
        pltpu.sync_copy(head_hbm, hidx_v)
        pltpu.sync_copy(tail_hbm, tidx_v)

        def stage_low(i, carry):
            sl = pl.ds(i * LANES, LANES)
            h = hidx_v[sl]
            t = tidx_v[sl]
            hm = h < SPLIT
            tm = t < SPLIT
            hv = plsc.load_gather(lo_v, [jnp.where(hm, h, 0)])
            tv = plsc.load_gather(lo_v, [jnp.where(tm, t, 0)])
            hval_v[sl] = jnp.where(hm, hv, 0.0)
            tval_v[sl] = jnp.where(tm, tv, 0.0)
            return carry

        def merge_high(i, acc):
            sl = pl.ds(i * LANES, LANES)
            h = hidx_v[sl]
            t = tidx_v[sl]
            hm = h >= SPLIT
            tm = t >= SPLIT
            hv = plsc.load_gather(hi_v, [jnp.where(hm, h - SPLIT, 0)])
            tv = plsc.load_gather(hi_v, [jnp.where(tm, t - SPLIT, 0)])
            hfull = jnp.where(hm, hv, hval_v[sl])
            tfull = jnp.where(tm, tv, tval_v[sl])
            d = hfull - tfull
            return acc + d * d

        acc = jnp.zeros((LANES,), jnp.float32)
        d_lo.wait()
        lax.fori_loop(0, N_CHUNKS, stage_low, jnp.int32(0))
        d_lo2 = pltpu.async_copy(tab_hbm.at[c1, pl.ds(0, SPLIT)], lo_v, sem_a)
        d_hi.wait()
        acc = lax.fori_loop(0, N_CHUNKS, merge_high, acc)
        d_hi2 = pltpu.async_copy(tab_hbm.at[c1, pl.ds(SPLIT, REST)], hi_v, sem_b)
        d_lo2.wait()
        lax.fori_loop(0, N_CHUNKS, stage_low, jnp.int32(0))
        d_hi2.wait()
        acc = lax.fori_loop(0, N_CHUNKS, merge_high, acc)
        acc_v[...] = acc
        pltpu.sync_copy(acc_v, out_hbm.at[wid])

    return k(table_t, head, tail)


def kernel(head, tail, table):
    partials = _sc_col_partials(table.T, head.astype(jnp.int32),
                                tail.astype(jnp.int32))
    return jnp.sqrt(jnp.sum(partials))


# speedup vs baseline: 2.4103x; 1.0629x over previous
import functools

import jax
import jax.numpy as jnp
from jax import lax
from jax.experimental import pallas as pl
from jax.experimental.pallas import tpu as pltpu
from jax.experimental.pallas import tpu_sc as plsc

VOCAB = 100000
EMBED_DIM = 64
BATCH = 4096
NC = 2
NS = 16
NW = NC * NS
COLS_PER_W = EMBED_DIM // NW
LANES = 16
N_CHUNKS = BATCH // LANES


def _sc_col_partials(table_t, head, tail):
    mesh = plsc.VectorSubcoreMesh(core_axis_name="c", subcore_axis_name="s",
                                  num_cores=NC, num_subcores=NS)

    @functools.partial(
        pl.kernel,
        out_type=jax.ShapeDtypeStruct((NW, LANES), jnp.float32),
        mesh=mesh,
        compiler_params=pltpu.CompilerParams(use_tc_tiling_on_sc=True,
                                             needs_layout_passes=False),
        scratch_types=[
            pltpu.VMEM((BATCH,), jnp.int32),
            pltpu.VMEM((BATCH,), jnp.int32),
            pltpu.VMEM((VOCAB,), jnp.float32),
            pltpu.VMEM((LANES,), jnp.float32),
            pltpu.SemaphoreType.DMA,
        ],
    )
    def k(tab_hbm, head_hbm, tail_hbm, out_hbm, hidx_v, tidx_v, col_v, acc_v,
          sem):
        wid = lax.axis_index("s") * NC + lax.axis_index("c")
        c0 = wid * COLS_PER_W
        d0 = pltpu.async_copy(tab_hbm.at[c0], col_v, sem)
        pltpu.sync_copy(head_hbm, hidx_v)
        pltpu.sync_copy(tail_hbm, tidx_v)

        def one_chunk(i, acc):
            hvec = plsc.load_gather(col_v, [hidx_v[pl.ds(i * LANES, LANES)]])
            tvec = plsc.load_gather(col_v, [tidx_v[pl.ds(i * LANES, LANES)]])
            d = hvec - tvec
            return acc + d * d

        d0.wait()
        acc = lax.fori_loop(0, N_CHUNKS, one_chunk,
                            jnp.zeros((LANES,), jnp.float32))
        for p in range(1, COLS_PER_W):
            pltpu.sync_copy(tab_hbm.at[c0 + p], col_v)
            acc = lax.fori_loop(0, N_CHUNKS, one_chunk, acc)
        acc_v[...] = acc
        pltpu.sync_copy(acc_v, out_hbm.at[wid])

    return k(table_t, head, tail)


def kernel(head, tail, table):
    partials = _sc_col_partials(table.T, head.astype(jnp.int32),
                                tail.astype(jnp.int32))
    return jnp.sqrt(jnp.sum(partials))


# speedup vs baseline: 2.4703x; 1.0249x over previous
import functools

import jax
import jax.numpy as jnp
from jax import lax
from jax.experimental import pallas as pl
from jax.experimental.pallas import tpu as pltpu
from jax.experimental.pallas import tpu_sc as plsc

VOCAB = 100000
EMBED_DIM = 64
BATCH = 4096
NC = 2
NS = 16
NW = NC * NS
COLS_PER_W = EMBED_DIM // NW
LANES = 16
N_CHUNKS = BATCH // LANES


def _sc_col_partials(table_t, head, tail):
    mesh = plsc.VectorSubcoreMesh(core_axis_name="c", subcore_axis_name="s",
                                  num_cores=NC, num_subcores=NS)

    @functools.partial(
        pl.kernel,
        out_type=jax.ShapeDtypeStruct((NW, LANES), jnp.float32),
        mesh=mesh,
        compiler_params=pltpu.CompilerParams(use_tc_tiling_on_sc=True,
                                             needs_layout_passes=False),
        scratch_types=[
            pltpu.VMEM((BATCH,), jnp.int32),
            pltpu.VMEM((BATCH,), jnp.int32),
            pltpu.VMEM((VOCAB,), jnp.float32),
            pltpu.VMEM((LANES,), jnp.float32),
            pltpu.SemaphoreType.DMA,
        ],
    )
    def k(tab_hbm, head_hbm, tail_hbm, out_hbm, hidx_v, tidx_v, col_v, acc_v,
          sem):
        wid = lax.axis_index("s") * NC + lax.axis_index("c")
        c0 = wid * COLS_PER_W
        d0 = pltpu.async_copy(tab_hbm.at[c0], col_v, sem)
        pltpu.sync_copy(head_hbm, hidx_v)
        pltpu.sync_copy(tail_hbm, tidx_v)

        def one_chunk(i, accs):
            a0, a1 = accs
            h0 = plsc.load_gather(col_v, [hidx_v[pl.ds(i * 2 * LANES, LANES)]])
            t0 = plsc.load_gather(col_v, [tidx_v[pl.ds(i * 2 * LANES, LANES)]])
            h1 = plsc.load_gather(
                col_v, [hidx_v[pl.ds((i * 2 + 1) * LANES, LANES)]])
            t1 = plsc.load_gather(
                col_v, [tidx_v[pl.ds((i * 2 + 1) * LANES, LANES)]])
            d0_ = h0 - t0
            d1_ = h1 - t1
            return (a0 + d0_ * d0_, a1 + d1_ * d1_)

        zero = jnp.zeros((LANES,), jnp.float32)
        d0.wait()
        accs = lax.fori_loop(0, N_CHUNKS // 2, one_chunk, (zero, zero))
        for p in range(1, COLS_PER_W):
            pltpu.sync_copy(tab_hbm.at[c0 + p], col_v)
            accs = lax.fori_loop(0, N_CHUNKS // 2, one_chunk, accs)
        acc_v[...] = accs[0] + accs[1]
        pltpu.sync_copy(acc_v, out_hbm.at[wid])

    return k(table_t, head, tail)


def kernel(head, tail, table):
    partials = _sc_col_partials(table.T, head.astype(jnp.int32),
                                tail.astype(jnp.int32))
    return jnp.sqrt(jnp.sum(partials))
